# baseline (speedup 1.0000x reference)
import functools

import jax
import jax.numpy as jnp
from jax import lax
from jax.experimental import pallas as pl
from jax.experimental.pallas import tpu as pltpu
from jax.experimental.pallas import tpu_sc as plsc

B, F, D = 16384, 26, 16
FD = F * D
FIELD_DIM = 40000
V = F * FIELD_DIM
K = 13
EPS = 1e-5
BD = B * D

NW = 32
PER_W = B * F // NW
CHUNK = 3328
NCHUNK = PER_W // CHUNK

RB = 2048
GRID = B // RB


TCOLS = 2048
TROWS = TCOLS // 8
NFULL = (V // 8) // TROWS
TAILC = V - NFULL * TCOLS
TAILR = TAILC // 8
NSLOTS = NFULL + 1


def _sc_transpose(emb_t):
    mesh = plsc.VectorSubcoreMesh(core_axis_name="c", subcore_axis_name="s")

    @functools.partial(
        pl.kernel,
        mesh=mesh,
        compiler_params=pltpu.CompilerParams(needs_layout_passes=False),
        out_type=jax.ShapeDtypeStruct((V // 8, 128), jnp.float32),
        scratch_types=[
            pltpu.VMEM((D, TCOLS), jnp.float32),
            pltpu.VMEM((TROWS, 128), jnp.float32),
        ],
    )
    def k(src_hbm, out_hbm, in_v, out_v):
        wid = lax.axis_index("s") * 2 + lax.axis_index("c")
        i16 = lax.broadcasted_iota(jnp.int32, (16,), 0)

        def do_chunk(c, ncols, nrows):
            pltpu.sync_copy(src_hbm.at[:, pl.ds(c * TCOLS, ncols)],
                            in_v.at[:, pl.ds(0, ncols)])

            @plsc.parallel_loop(0, nrows, step=1, unroll=8)
            def tr_body(j8):
                r8 = jnp.full((16,), j8, jnp.int32)
                for jj in range(8):
                    col = jnp.full((16,), j8 * 8 + jj, jnp.int32)
                    vals = plsc.load_gather(in_v, [i16, col])
                    plsc.store_scatter(out_v, [r8, i16 + jj * 16], vals)
            pltpu.sync_copy(out_v.at[pl.ds(0, nrows)],
                            out_hbm.at[pl.ds(c * TROWS, nrows)])

        def body(i, carry):
            c = wid + i * NW

            @pl.when(c < NFULL)
            def _():
                do_chunk(c, TCOLS, TROWS)

            @pl.when(c == NFULL)
            def _():
                do_chunk(c, TAILC, TAILR)

            return carry

        lax.fori_loop(0, pl.cdiv(NSLOTS, NW), body, 0)

    return k(emb_t)


def _sc_gather(xo_flat, emb_table, lin_flat):
    mesh = plsc.VectorSubcoreMesh(core_axis_name="c", subcore_axis_name="s")

    @functools.partial(
        pl.kernel,
        mesh=mesh,
        compiler_params=pltpu.CompilerParams(use_tc_tiling_on_sc=False),
        out_type=[
            jax.ShapeDtypeStruct((B * F, D), jnp.float32),
            jax.ShapeDtypeStruct((B * F,), jnp.float32),
        ],
        scratch_types=[
            pltpu.VMEM((CHUNK,), jnp.int32),
            pltpu.VMEM((CHUNK, D), jnp.float32),
            pltpu.VMEM((CHUNK,), jnp.float32),
            pltpu.SemaphoreType.DMA,
            pltpu.SemaphoreType.DMA,
        ],
    )
    def k(xo_hbm, emb_hbm, lin_hbm, out_emb, out_lin, idx_v, rows_v, linv_v,
          sem1, sem2):
        wid = lax.axis_index("s") * 2 + lax.axis_index("c")
        wbase = wid * PER_W

        def body(i, carry):
            base = wbase + i * CHUNK
            pltpu.sync_copy(xo_hbm.at[pl.ds(base, CHUNK)], idx_v)
            cp1 = pltpu.async_copy(emb_hbm.at[idx_v], rows_v, sem1)
            cp2 = pltpu.async_copy(lin_hbm.at[idx_v], linv_v, sem2)
            cp1.wait()
            cp2.wait()
            pltpu.sync_copy(rows_v, out_emb.at[pl.ds(base, CHUNK)])
            pltpu.sync_copy(linv_v, out_lin.at[pl.ds(base, CHUNK)])
            return carry

        lax.fori_loop(0, NCHUNK, body, 0)

    return k(xo_flat, emb_table, lin_flat)


def _stats_body(e_ref, o_ref):
    @pl.when(pl.program_id(0) == 0)
    def _():
        o_ref[...] = jnp.zeros_like(o_ref)

    e = e_ref[...]
    o_ref[0:1, :] += jnp.sum(e, axis=0, keepdims=True)
    o_ref[1:2, :] += jnp.sum(e * e, axis=0, keepdims=True)


def _stats_call(e2d):
    return pl.pallas_call(
        _stats_body,
        grid=(GRID,),
        in_specs=[pl.BlockSpec((RB, FD), lambda i: (i, 0))],
        out_specs=pl.BlockSpec((2, FD), lambda i: (0, 0)),
        out_shape=jax.ShapeDtypeStruct((2, FD), jnp.float32),
    )(e2d)


def _fold_mats():
    r26 = lax.broadcasted_iota(jnp.int32, (F, FD), 0)
    c416 = lax.broadcasted_iota(jnp.int32, (F, FD), 1)
    E = (c416 // D == r26).astype(jnp.float32)
    r416 = lax.broadcasted_iota(jnp.int32, (FD, F), 0)
    c26 = lax.broadcasted_iota(jnp.int32, (FD, F), 1)
    ET = (r416 // D == c26).astype(jnp.float32)
    return E, ET


def _bn1_affine(stats_ref, g1_ref, b1_ref, E, ET):
    s = stats_ref[0:1, :]
    ss = stats_ref[1:2, :]
    mean = jnp.dot(s, ET) / BD
    ex2 = jnp.dot(ss, ET) / BD
    var = ex2 - mean * mean
    a26 = g1_ref[...] / jnp.sqrt(var + EPS)
    b26 = b1_ref[...] - mean * a26
    a416 = jnp.dot(a26, E)
    b416 = jnp.dot(b26, E)
    return a416, b416


def _mm_body(e_ref, stats_ref, wc_ref, bc_ref, g1_ref, b1_ref,
             ht_ref, hstats_ref):
    E, ET = _fold_mats()
    a416, b416 = _bn1_affine(stats_ref, g1_ref, b1_ref, E, ET)
    ebn = a416 * e_ref[...] + b416
    h = lax.dot_general(wc_ref[...], ebn,
                        (((1,), (1,)), ((), ())),
                        preferred_element_type=jnp.float32)
    h = h + bc_ref[...]
    ht_ref[...] = h

    @pl.when(pl.program_id(0) == 0)
    def _():
        hstats_ref[...] = jnp.zeros_like(hstats_ref)

    hstats_ref[:, 0:1] += jnp.sum(h, axis=1, keepdims=True)
    hstats_ref[:, 1:2] += jnp.sum(h * h, axis=1, keepdims=True)


def _mm_call(e2d, stats, Wc, bc_c, g1_r, b1_r):
    return pl.pallas_call(
        _mm_body,
        grid=(GRID,),
        in_specs=[
            pl.BlockSpec((RB, FD), lambda i: (i, 0)),
            pl.BlockSpec((2, FD), lambda i: (0, 0)),
            pl.BlockSpec((F, FD), lambda i: (0, 0)),
            pl.BlockSpec((F, 1), lambda i: (0, 0)),
            pl.BlockSpec((1, F), lambda i: (0, 0)),
            pl.BlockSpec((1, F), lambda i: (0, 0)),
        ],
        out_specs=[
            pl.BlockSpec((F, RB), lambda i: (0, i)),
            pl.BlockSpec((F, 2), lambda i: (0, 0)),
        ],
        out_shape=[
            jax.ShapeDtypeStruct((F, B), jnp.float32),
            jax.ShapeDtypeStruct((F, 2), jnp.float32),
        ],
    )(e2d, stats, Wc, bc_c, g1_r, b1_r)


def _final_body(e_ref, ht_ref, hstats_ref, stats_ref, lin_ref,
                g1_ref, b1_ref, g2_ref, b2_ref, lb_ref, out_ref):
    E, ET = _fold_mats()
    a416, b416 = _bn1_affine(stats_ref, g1_ref, b1_ref, E, ET)

    m2 = hstats_ref[:, 0:1] / B
    v2 = hstats_ref[:, 1:2] / B - m2 * m2
    h = ht_ref[...]
    w = g2_ref[...] * (h - m2) / jnp.sqrt(v2 + EPS) + b2_ref[...]
    w = jnp.maximum(w, 0.0)

    row_id = lax.broadcasted_iota(jnp.int32, (F, RB), 0)
    rank = jnp.zeros(w.shape, jnp.float32)
    for j in range(F):
        wj = w[j:j + 1, :]
        gt = (wj > w).astype(jnp.float32)
        eq = jnp.where((wj == w) & (row_id > j), 1.0, 0.0)
        rank = rank + gt + eq
    sel = rank < K
    wsel = jnp.where(sel, w, 0.0)
    ssum = jnp.sum(wsel, axis=0, keepdims=True)
    maskT = wsel / ssum

    mrep = lax.dot_general(maskT, E, (((0,), (0,)), ((), ())),
                           preferred_element_type=jnp.float32)
    ebn = a416 * e_ref[...] + b416
    em = ebn * mrep
    rS = lax.broadcasted_iota(jnp.int32, (FD, D), 0)
    cS = lax.broadcasted_iota(jnp.int32, (FD, D), 1)
    S = (rS % D == cS).astype(jnp.float32)
    s16 = jnp.dot(em, S, preferred_element_type=jnp.float32)
    fm = 0.5 * (jnp.sum(s16 * s16, axis=1, keepdims=True)
                - jnp.sum(em * em, axis=1, keepdims=True))

    linv = jnp.sum(lin_ref[...], axis=1, keepdims=True) + lb_ref[...]
    out_ref[...] = jax.nn.sigmoid(linv + fm)


def _final_call(e2d, ht, hstats, stats, lin2d, g1_r, b1_r, g2_c, b2_c, lb):
    return pl.pallas_call(
        _final_body,
        grid=(GRID,),
        in_specs=[
            pl.BlockSpec((RB, FD), lambda i: (i, 0)),
            pl.BlockSpec((F, RB), lambda i: (0, i)),
            pl.BlockSpec((F, 2), lambda i: (0, 0)),
            pl.BlockSpec((2, FD), lambda i: (0, 0)),
            pl.BlockSpec((RB, F), lambda i: (i, 0)),
            pl.BlockSpec((1, F), lambda i: (0, 0)),
            pl.BlockSpec((1, F), lambda i: (0, 0)),
            pl.BlockSpec((F, 1), lambda i: (0, 0)),
            pl.BlockSpec((F, 1), lambda i: (0, 0)),
            pl.BlockSpec((1, 1), lambda i: (0, 0)),
        ],
        out_specs=pl.BlockSpec((RB, 1), lambda i: (i, 0)),
        out_shape=jax.ShapeDtypeStruct((B, 1), jnp.float32),
    )(e2d, ht, hstats, stats, lin2d, g1_r, b1_r, g2_c, b2_c, lb)


def kernel(x, emb_table, lin_table, lin_bias, bn1_gamma, bn1_beta, Wc, bc,
           bn2_gamma, bn2_beta):
    offs = jnp.arange(F, dtype=jnp.int32) * FIELD_DIM
    xo = (x + offs[None, :]).reshape(B * F)
    emb_lin = _sc_transpose(emb_table.T).reshape(V, D)
    emb_rows, lin_vals = _sc_gather(xo, emb_lin, lin_table.reshape(V))
    e2d = emb_rows.reshape(B, FD)
    lin2d = lin_vals.reshape(B, F)

    stats = _stats_call(e2d)
    ht, hstats = _mm_call(e2d, stats, Wc,
                          bc.reshape(F, 1),
                          bn1_gamma.reshape(1, F), bn1_beta.reshape(1, F))
    out2d = _final_call(e2d, ht, hstats, stats, lin2d,
                        bn1_gamma.reshape(1, F), bn1_beta.reshape(1, F),
                        bn2_gamma.reshape(F, 1), bn2_beta.reshape(F, 1),
                        lin_bias.reshape(1, 1))
    return out2d.reshape(B)


# speedup vs baseline: 1.1715x; 1.1715x over previous
import functools

import jax
import jax.numpy as jnp
from jax import lax
from jax.experimental import pallas as pl
from jax.experimental.pallas import tpu as pltpu
from jax.experimental.pallas import tpu_sc as plsc

B, F, D = 16384, 26, 16
FD = F * D
FIELD_DIM = 40000
V = F * FIELD_DIM
K = 13
EPS = 1e-5
BD = B * D

NW = 32
PER_W = B * F // NW
CHUNK = 3328
NCHUNK = PER_W // CHUNK

RB = 2048
GRID = B // RB


TCOLS = 2048
TROWS = TCOLS // 8
NFULL = (V // 8) // TROWS
TAILC = V - NFULL * TCOLS
TAILR = TAILC // 8
NSLOTS = NFULL + 1


TWORDS = TROWS * 128
MAXI = -(-NFULL // NW) + 1


def _sc_transpose(emb_t):
    mesh = plsc.VectorSubcoreMesh(core_axis_name="c", subcore_axis_name="s")

    @functools.partial(
        pl.kernel,
        mesh=mesh,
        compiler_params=pltpu.CompilerParams(needs_layout_passes=False),
        out_type=jax.ShapeDtypeStruct((V * D,), jnp.float32),
        scratch_types=[
            pltpu.VMEM((D, TCOLS), jnp.float32),
            pltpu.VMEM((D, TCOLS), jnp.float32),
            pltpu.VMEM((TWORDS,), jnp.float32),
            pltpu.VMEM((TWORDS,), jnp.float32),
            pltpu.SemaphoreType.DMA,
            pltpu.SemaphoreType.DMA,
            pltpu.SemaphoreType.DMA,
            pltpu.SemaphoreType.DMA,
        ],
    )
    def k(src_hbm, out_hbm, in0, in1, ou0, ou1, si0, si1, so0, so1):
        wid = lax.axis_index("s") * 2 + lax.axis_index("c")
        i16 = lax.broadcasted_iota(jnp.int32, (16,), 0)
        ins, ous = (in0, in1), (ou0, ou1)
        sis, sos = (si0, si1), (so0, so1)

        def in_start(c, p):
            return pltpu.async_copy(src_hbm.at[:, pl.ds(c * TCOLS, TCOLS)],
                                    ins[p], sis[p])

        def transpose_rows(in_p, ou_p, nrows):
            @plsc.parallel_loop(0, nrows, step=1, unroll=4)
            def tr_body(j8):
                for jj in range(8):
                    col = jnp.full((16,), j8 * 8 + jj, jnp.int32)
                    vals = plsc.load_gather(in_p, [i16, col])
                    ou_p[pl.ds(j8 * 128 + jj * 16, 16)] = vals

        def step(i, c, p):
            @pl.when(c + NW < NFULL)
            def _():
                in_start(c + NW, 1 - p)

            pltpu.make_async_copy(src_hbm.at[:, pl.ds(c * TCOLS, TCOLS)],
                                  ins[p], sis[p]).wait()

            @pl.when(i >= 2)
            def _():
                pltpu.make_async_copy(ous[p], out_hbm.at[pl.ds(0, TWORDS)],
                                      sos[p]).wait()

            transpose_rows(ins[p], ous[p], TROWS)
            pltpu.async_copy(ous[p], out_hbm.at[pl.ds(c * TWORDS, TWORDS)],
                             sos[p])

        @pl.when(wid < NFULL)
        def _():
            in_start(wid, 0)

        def body(i, carry):
            c = wid + i * NW

            @pl.when(c < NFULL)
            def _():
                @pl.when(i % 2 == 0)
                def _():
                    step(i, c, 0)

                @pl.when(i % 2 == 1)
                def _():
                    step(i, c, 1)

            return carry

        lax.fori_loop(0, MAXI - 1, body, 0)

        for p in range(2):
            pltpu.make_async_copy(ous[p], out_hbm.at[pl.ds(0, TWORDS)],
                                  sos[p]).wait()

        @pl.when(wid == NFULL % NW)
        def _():
            pltpu.sync_copy(src_hbm.at[:, pl.ds(NFULL * TCOLS, TAILC)],
                            in0.at[:, pl.ds(0, TAILC)])
            transpose_rows(in0, ou0, TAILR)
            pltpu.sync_copy(ou0.at[pl.ds(0, TAILR * 128)],
                            out_hbm.at[pl.ds(NFULL * TWORDS, TAILR * 128)])

    return k(emb_t)


def _sc_gather(xo_flat, emb_table, lin_flat):
    mesh = plsc.VectorSubcoreMesh(core_axis_name="c", subcore_axis_name="s")

    @functools.partial(
        pl.kernel,
        mesh=mesh,
        compiler_params=pltpu.CompilerParams(use_tc_tiling_on_sc=False),
        out_type=[
            jax.ShapeDtypeStruct((B * F, D), jnp.float32),
            jax.ShapeDtypeStruct((B * F,), jnp.float32),
        ],
        scratch_types=[
            pltpu.VMEM((CHUNK,), jnp.int32),
            pltpu.VMEM((CHUNK, D), jnp.float32),
            pltpu.VMEM((CHUNK,), jnp.float32),
            pltpu.SemaphoreType.DMA,
            pltpu.SemaphoreType.DMA,
        ],
    )
    def k(xo_hbm, emb_hbm, lin_hbm, out_emb, out_lin, idx_v, rows_v, linv_v,
          sem1, sem2):
        wid = lax.axis_index("s") * 2 + lax.axis_index("c")
        wbase = wid * PER_W

        def body(i, carry):
            base = wbase + i * CHUNK
            pltpu.sync_copy(xo_hbm.at[pl.ds(base, CHUNK)], idx_v)
            cp1 = pltpu.async_copy(emb_hbm.at[idx_v], rows_v, sem1)
            cp2 = pltpu.async_copy(lin_hbm.at[idx_v], linv_v, sem2)
            cp1.wait()
            cp2.wait()
            pltpu.sync_copy(rows_v, out_emb.at[pl.ds(base, CHUNK)])
            pltpu.sync_copy(linv_v, out_lin.at[pl.ds(base, CHUNK)])
            return carry

        lax.fori_loop(0, NCHUNK, body, 0)

    return k(xo_flat, emb_table, lin_flat)


def _stats_body(e_ref, o_ref):
    @pl.when(pl.program_id(0) == 0)
    def _():
        o_ref[...] = jnp.zeros_like(o_ref)

    e = e_ref[...]
    o_ref[0:1, :] += jnp.sum(e, axis=0, keepdims=True)
    o_ref[1:2, :] += jnp.sum(e * e, axis=0, keepdims=True)


def _stats_call(e2d):
    return pl.pallas_call(
        _stats_body,
        grid=(GRID,),
        in_specs=[pl.BlockSpec((RB, FD), lambda i: (i, 0))],
        out_specs=pl.BlockSpec((2, FD), lambda i: (0, 0)),
        out_shape=jax.ShapeDtypeStruct((2, FD), jnp.float32),
    )(e2d)


def _fold_mats():
    r26 = lax.broadcasted_iota(jnp.int32, (F, FD), 0)
    c416 = lax.broadcasted_iota(jnp.int32, (F, FD), 1)
    E = (c416 // D == r26).astype(jnp.float32)
    r416 = lax.broadcasted_iota(jnp.int32, (FD, F), 0)
    c26 = lax.broadcasted_iota(jnp.int32, (FD, F), 1)
    ET = (r416 // D == c26).astype(jnp.float32)
    return E, ET


def _bn1_affine(stats_ref, g1_ref, b1_ref, E, ET):
    s = stats_ref[0:1, :]
    ss = stats_ref[1:2, :]
    mean = jnp.dot(s, ET) / BD
    ex2 = jnp.dot(ss, ET) / BD
    var = ex2 - mean * mean
    a26 = g1_ref[...] / jnp.sqrt(var + EPS)
    b26 = b1_ref[...] - mean * a26
    a416 = jnp.dot(a26, E)
    b416 = jnp.dot(b26, E)
    return a416, b416


def _mm_body(e_ref, stats_ref, wc_ref, bc_ref, g1_ref, b1_ref,
             ht_ref, hstats_ref):
    E, ET = _fold_mats()
    a416, b416 = _bn1_affine(stats_ref, g1_ref, b1_ref, E, ET)
    ebn = a416 * e_ref[...] + b416
    h = lax.dot_general(wc_ref[...], ebn,
                        (((1,), (1,)), ((), ())),
                        preferred_element_type=jnp.float32)
    h = h + bc_ref[...]
    ht_ref[...] = h

    @pl.when(pl.program_id(0) == 0)
    def _():
        hstats_ref[...] = jnp.zeros_like(hstats_ref)

    hstats_ref[:, 0:1] += jnp.sum(h, axis=1, keepdims=True)
    hstats_ref[:, 1:2] += jnp.sum(h * h, axis=1, keepdims=True)


def _mm_call(e2d, stats, Wc, bc_c, g1_r, b1_r):
    return pl.pallas_call(
        _mm_body,
        grid=(GRID,),
        in_specs=[
            pl.BlockSpec((RB, FD), lambda i: (i, 0)),
            pl.BlockSpec((2, FD), lambda i: (0, 0)),
            pl.BlockSpec((F, FD), lambda i: (0, 0)),
            pl.BlockSpec((F, 1), lambda i: (0, 0)),
            pl.BlockSpec((1, F), lambda i: (0, 0)),
            pl.BlockSpec((1, F), lambda i: (0, 0)),
        ],
        out_specs=[
            pl.BlockSpec((F, RB), lambda i: (0, i)),
            pl.BlockSpec((F, 2), lambda i: (0, 0)),
        ],
        out_shape=[
            jax.ShapeDtypeStruct((F, B), jnp.float32),
            jax.ShapeDtypeStruct((F, 2), jnp.float32),
        ],
    )(e2d, stats, Wc, bc_c, g1_r, b1_r)


def _final_body(e_ref, ht_ref, hstats_ref, stats_ref, lin_ref,
                g1_ref, b1_ref, g2_ref, b2_ref, lb_ref, out_ref):
    E, ET = _fold_mats()
    a416, b416 = _bn1_affine(stats_ref, g1_ref, b1_ref, E, ET)

    m2 = hstats_ref[:, 0:1] / B
    v2 = hstats_ref[:, 1:2] / B - m2 * m2
    h = ht_ref[...]
    w = g2_ref[...] * (h - m2) / jnp.sqrt(v2 + EPS) + b2_ref[...]
    w = jnp.maximum(w, 0.0)

    row_id = lax.broadcasted_iota(jnp.int32, (F, RB), 0)
    rank = jnp.zeros(w.shape, jnp.float32)
    for j in range(F):
        wj = w[j:j + 1, :]
        gt = (wj > w).astype(jnp.float32)
        eq = jnp.where((wj == w) & (row_id > j), 1.0, 0.0)
        rank = rank + gt + eq
    sel = rank < K
    wsel = jnp.where(sel, w, 0.0)
    ssum = jnp.sum(wsel, axis=0, keepdims=True)
    maskT = wsel / ssum

    mrep = lax.dot_general(maskT, E, (((0,), (0,)), ((), ())),
                           preferred_element_type=jnp.float32)
    ebn = a416 * e_ref[...] + b416
    em = ebn * mrep
    rS = lax.broadcasted_iota(jnp.int32, (FD, D), 0)
    cS = lax.broadcasted_iota(jnp.int32, (FD, D), 1)
    S = (rS % D == cS).astype(jnp.float32)
    s16 = jnp.dot(em, S, preferred_element_type=jnp.float32)
    fm = 0.5 * (jnp.sum(s16 * s16, axis=1, keepdims=True)
                - jnp.sum(em * em, axis=1, keepdims=True))

    linv = jnp.sum(lin_ref[...], axis=1, keepdims=True) + lb_ref[...]
    out_ref[...] = jax.nn.sigmoid(linv + fm)


def _final_call(e2d, ht, hstats, stats, lin2d, g1_r, b1_r, g2_c, b2_c, lb):
    return pl.pallas_call(
        _final_body,
        grid=(GRID,),
        in_specs=[
            pl.BlockSpec((RB, FD), lambda i: (i, 0)),
            pl.BlockSpec((F, RB), lambda i: (0, i)),
            pl.BlockSpec((F, 2), lambda i: (0, 0)),
            pl.BlockSpec((2, FD), lambda i: (0, 0)),
            pl.BlockSpec((RB, F), lambda i: (i, 0)),
            pl.BlockSpec((1, F), lambda i: (0, 0)),
            pl.BlockSpec((1, F), lambda i: (0, 0)),
            pl.BlockSpec((F, 1), lambda i: (0, 0)),
            pl.BlockSpec((F, 1), lambda i: (0, 0)),
            pl.BlockSpec((1, 1), lambda i: (0, 0)),
        ],
        out_specs=pl.BlockSpec((RB, 1), lambda i: (i, 0)),
        out_shape=jax.ShapeDtypeStruct((B, 1), jnp.float32),
    )(e2d, ht, hstats, stats, lin2d, g1_r, b1_r, g2_c, b2_c, lb)


def kernel(x, emb_table, lin_table, lin_bias, bn1_gamma, bn1_beta, Wc, bc,
           bn2_gamma, bn2_beta):
    offs = jnp.arange(F, dtype=jnp.int32) * FIELD_DIM
    xo = (x + offs[None, :]).reshape(B * F)
    emb_lin = _sc_transpose(emb_table.T).reshape(V, D)
    emb_rows, lin_vals = _sc_gather(xo, emb_lin, lin_table.reshape(V))
    e2d = emb_rows.reshape(B, FD)
    lin2d = lin_vals.reshape(B, F)

    stats = _stats_call(e2d)
    ht, hstats = _mm_call(e2d, stats, Wc,
                          bc.reshape(F, 1),
                          bn1_gamma.reshape(1, F), bn1_beta.reshape(1, F))
    out2d = _final_call(e2d, ht, hstats, stats, lin2d,
                        bn1_gamma.reshape(1, F), bn1_beta.reshape(1, F),
                        bn2_gamma.reshape(F, 1), bn2_beta.reshape(F, 1),
                        lin_bias.reshape(1, 1))
    return out2d.reshape(B)


# speedup vs baseline: 1.9992x; 1.7066x over previous
import functools

import jax
import jax.numpy as jnp
from jax import lax
from jax.experimental import pallas as pl
from jax.experimental.pallas import tpu as pltpu
from jax.experimental.pallas import tpu_sc as plsc

B, F, D = 16384, 26, 16
FD = F * D
FIELD_DIM = 40000
V = F * FIELD_DIM
K = 13
EPS = 1e-5
BD = B * D

NW = 32
PER_W = B * F // NW
CHUNK = 3328
NCHUNK = PER_W // CHUNK

RB = 2048
GRID = B // RB


TCOLS = 2048
TROWS = TCOLS // 8
NFULL = (V // 8) // TROWS
TAILC = V - NFULL * TCOLS
TAILR = TAILC // 8
NSLOTS = NFULL + 1


TWORDS = TROWS * 128
MAXI = -(-NFULL // NW) + 1


def _sc_transpose(emb_t):
    mesh = plsc.VectorSubcoreMesh(core_axis_name="c", subcore_axis_name="s")

    @functools.partial(
        pl.kernel,
        mesh=mesh,
        compiler_params=pltpu.CompilerParams(needs_layout_passes=False),
        out_type=jax.ShapeDtypeStruct((V * D,), jnp.float32),
        scratch_types=[
            pltpu.VMEM((D, TCOLS), jnp.float32),
            pltpu.VMEM((D, TCOLS), jnp.float32),
            pltpu.VMEM((TWORDS,), jnp.float32),
            pltpu.VMEM((TWORDS,), jnp.float32),
            pltpu.SemaphoreType.DMA,
            pltpu.SemaphoreType.DMA,
            pltpu.SemaphoreType.DMA,
            pltpu.SemaphoreType.DMA,
        ],
    )
    def k(src_hbm, out_hbm, in0, in1, ou0, ou1, si0, si1, so0, so1):
        wid = lax.axis_index("s") * 2 + lax.axis_index("c")
        i16 = lax.broadcasted_iota(jnp.int32, (16,), 0)
        ins, ous = (in0, in1), (ou0, ou1)
        sis, sos = (si0, si1), (so0, so1)

        def in_start(c, p):
            return pltpu.async_copy(src_hbm.at[:, pl.ds(c * TCOLS, TCOLS)],
                                    ins[p], sis[p])

        def transpose_rows(in_p, ou_p, nrows):
            c16 = i16 * 16

            @plsc.parallel_loop(0, nrows // 2, step=1, unroll=2)
            def tr_body(g):
                for d in range(16):
                    vals = in_p[d, pl.ds(g * 16, 16)]
                    plsc.store_scatter(ou_p, [c16 + (g * 256 + d)], vals)

        def step(i, c, p):
            @pl.when(c + NW < NFULL)
            def _():
                in_start(c + NW, 1 - p)

            pltpu.make_async_copy(src_hbm.at[:, pl.ds(c * TCOLS, TCOLS)],
                                  ins[p], sis[p]).wait()

            @pl.when(i >= 2)
            def _():
                pltpu.make_async_copy(ous[p], out_hbm.at[pl.ds(0, TWORDS)],
                                      sos[p]).wait()

            transpose_rows(ins[p], ous[p], TROWS)
            pltpu.async_copy(ous[p], out_hbm.at[pl.ds(c * TWORDS, TWORDS)],
                             sos[p])

        @pl.when(wid < NFULL)
        def _():
            in_start(wid, 0)

        def body(i, carry):
            c = wid + i * NW

            @pl.when(c < NFULL)
            def _():
                @pl.when(i % 2 == 0)
                def _():
                    step(i, c, 0)

                @pl.when(i % 2 == 1)
                def _():
                    step(i, c, 1)

            return carry

        lax.fori_loop(0, MAXI - 1, body, 0)

        for p in range(2):
            pltpu.make_async_copy(ous[p], out_hbm.at[pl.ds(0, TWORDS)],
                                  sos[p]).wait()

        @pl.when(wid == NFULL % NW)
        def _():
            pltpu.sync_copy(src_hbm.at[:, pl.ds(NFULL * TCOLS, TAILC)],
                            in0.at[:, pl.ds(0, TAILC)])
            transpose_rows(in0, ou0, TAILR)
            pltpu.sync_copy(ou0.at[pl.ds(0, TAILR * 128)],
                            out_hbm.at[pl.ds(NFULL * TWORDS, TAILR * 128)])

    return k(emb_t)


def _sc_gather(xo_flat, emb_table, lin_flat):
    mesh = plsc.VectorSubcoreMesh(core_axis_name="c", subcore_axis_name="s")

    @functools.partial(
        pl.kernel,
        mesh=mesh,
        compiler_params=pltpu.CompilerParams(use_tc_tiling_on_sc=False),
        out_type=[
            jax.ShapeDtypeStruct((B * F, D), jnp.float32),
            jax.ShapeDtypeStruct((B * F,), jnp.float32),
        ],
        scratch_types=[
            pltpu.VMEM((CHUNK,), jnp.int32),
            pltpu.VMEM((CHUNK, D), jnp.float32),
            pltpu.VMEM((CHUNK,), jnp.float32),
            pltpu.SemaphoreType.DMA,
            pltpu.SemaphoreType.DMA,
        ],
    )
    def k(xo_hbm, emb_hbm, lin_hbm, out_emb, out_lin, idx_v, rows_v, linv_v,
          sem1, sem2):
        wid = lax.axis_index("s") * 2 + lax.axis_index("c")
        wbase = wid * PER_W

        def body(i, carry):
            base = wbase + i * CHUNK
            pltpu.sync_copy(xo_hbm.at[pl.ds(base, CHUNK)], idx_v)
            cp1 = pltpu.async_copy(emb_hbm.at[idx_v], rows_v, sem1)
            cp2 = pltpu.async_copy(lin_hbm.at[idx_v], linv_v, sem2)
            cp1.wait()
            cp2.wait()
            pltpu.sync_copy(rows_v, out_emb.at[pl.ds(base, CHUNK)])
            pltpu.sync_copy(linv_v, out_lin.at[pl.ds(base, CHUNK)])
            return carry

        lax.fori_loop(0, NCHUNK, body, 0)

    return k(xo_flat, emb_table, lin_flat)


def _stats_body(e_ref, o_ref):
    @pl.when(pl.program_id(0) == 0)
    def _():
        o_ref[...] = jnp.zeros_like(o_ref)

    e = e_ref[...]
    o_ref[0:1, :] += jnp.sum(e, axis=0, keepdims=True)
    o_ref[1:2, :] += jnp.sum(e * e, axis=0, keepdims=True)


def _stats_call(e2d):
    return pl.pallas_call(
        _stats_body,
        grid=(GRID,),
        in_specs=[pl.BlockSpec((RB, FD), lambda i: (i, 0))],
        out_specs=pl.BlockSpec((2, FD), lambda i: (0, 0)),
        out_shape=jax.ShapeDtypeStruct((2, FD), jnp.float32),
    )(e2d)


def _fold_mats():
    r26 = lax.broadcasted_iota(jnp.int32, (F, FD), 0)
    c416 = lax.broadcasted_iota(jnp.int32, (F, FD), 1)
    E = (c416 // D == r26).astype(jnp.float32)
    r416 = lax.broadcasted_iota(jnp.int32, (FD, F), 0)
    c26 = lax.broadcasted_iota(jnp.int32, (FD, F), 1)
    ET = (r416 // D == c26).astype(jnp.float32)
    return E, ET


def _bn1_affine(stats_ref, g1_ref, b1_ref, E, ET):
    s = stats_ref[0:1, :]
    ss = stats_ref[1:2, :]
    mean = jnp.dot(s, ET) / BD
    ex2 = jnp.dot(ss, ET) / BD
    var = ex2 - mean * mean
    a26 = g1_ref[...] / jnp.sqrt(var + EPS)
    b26 = b1_ref[...] - mean * a26
    a416 = jnp.dot(a26, E)
    b416 = jnp.dot(b26, E)
    return a416, b416


def _mm_body(e_ref, stats_ref, wc_ref, bc_ref, g1_ref, b1_ref,
             ht_ref, hstats_ref):
    E, ET = _fold_mats()
    a416, b416 = _bn1_affine(stats_ref, g1_ref, b1_ref, E, ET)
    ebn = a416 * e_ref[...] + b416
    h = lax.dot_general(wc_ref[...], ebn,
                        (((1,), (1,)), ((), ())),
                        preferred_element_type=jnp.float32)
    h = h + bc_ref[...]
    ht_ref[...] = h

    @pl.when(pl.program_id(0) == 0)
    def _():
        hstats_ref[...] = jnp.zeros_like(hstats_ref)

    hstats_ref[:, 0:1] += jnp.sum(h, axis=1, keepdims=True)
    hstats_ref[:, 1:2] += jnp.sum(h * h, axis=1, keepdims=True)


def _mm_call(e2d, stats, Wc, bc_c, g1_r, b1_r):
    return pl.pallas_call(
        _mm_body,
        grid=(GRID,),
        in_specs=[
            pl.BlockSpec((RB, FD), lambda i: (i, 0)),
            pl.BlockSpec((2, FD), lambda i: (0, 0)),
            pl.BlockSpec((F, FD), lambda i: (0, 0)),
            pl.BlockSpec((F, 1), lambda i: (0, 0)),
            pl.BlockSpec((1, F), lambda i: (0, 0)),
            pl.BlockSpec((1, F), lambda i: (0, 0)),
        ],
        out_specs=[
            pl.BlockSpec((F, RB), lambda i: (0, i)),
            pl.BlockSpec((F, 2), lambda i: (0, 0)),
        ],
        out_shape=[
            jax.ShapeDtypeStruct((F, B), jnp.float32),
            jax.ShapeDtypeStruct((F, 2), jnp.float32),
        ],
    )(e2d, stats, Wc, bc_c, g1_r, b1_r)


def _final_body(e_ref, ht_ref, hstats_ref, stats_ref, lin_ref,
                g1_ref, b1_ref, g2_ref, b2_ref, lb_ref, out_ref):
    E, ET = _fold_mats()
    a416, b416 = _bn1_affine(stats_ref, g1_ref, b1_ref, E, ET)

    m2 = hstats_ref[:, 0:1] / B
    v2 = hstats_ref[:, 1:2] / B - m2 * m2
    h = ht_ref[...]
    w = g2_ref[...] * (h - m2) / jnp.sqrt(v2 + EPS) + b2_ref[...]
    w = jnp.maximum(w, 0.0)

    row_id = lax.broadcasted_iota(jnp.int32, (F, RB), 0)
    rank = jnp.zeros(w.shape, jnp.float32)
    for j in range(F):
        wj = w[j:j + 1, :]
        gt = (wj > w).astype(jnp.float32)
        eq = jnp.where((wj == w) & (row_id > j), 1.0, 0.0)
        rank = rank + gt + eq
    sel = rank < K
    wsel = jnp.where(sel, w, 0.0)
    ssum = jnp.sum(wsel, axis=0, keepdims=True)
    maskT = wsel / ssum

    mrep = lax.dot_general(maskT, E, (((0,), (0,)), ((), ())),
                           preferred_element_type=jnp.float32)
    ebn = a416 * e_ref[...] + b416
    em = ebn * mrep
    rS = lax.broadcasted_iota(jnp.int32, (FD, D), 0)
    cS = lax.broadcasted_iota(jnp.int32, (FD, D), 1)
    S = (rS % D == cS).astype(jnp.float32)
    s16 = jnp.dot(em, S, preferred_element_type=jnp.float32)
    fm = 0.5 * (jnp.sum(s16 * s16, axis=1, keepdims=True)
                - jnp.sum(em * em, axis=1, keepdims=True))

    linv = jnp.sum(lin_ref[...], axis=1, keepdims=True) + lb_ref[...]
    out_ref[...] = jax.nn.sigmoid(linv + fm)


def _final_call(e2d, ht, hstats, stats, lin2d, g1_r, b1_r, g2_c, b2_c, lb):
    return pl.pallas_call(
        _final_body,
        grid=(GRID,),
        in_specs=[
            pl.BlockSpec((RB, FD), lambda i: (i, 0)),
            pl.BlockSpec((F, RB), lambda i: (0, i)),
            pl.BlockSpec((F, 2), lambda i: (0, 0)),
            pl.BlockSpec((2, FD), lambda i: (0, 0)),
            pl.BlockSpec((RB, F), lambda i: (i, 0)),
            pl.BlockSpec((1, F), lambda i: (0, 0)),
            pl.BlockSpec((1, F), lambda i: (0, 0)),
            pl.BlockSpec((F, 1), lambda i: (0, 0)),
            pl.BlockSpec((F, 1), lambda i: (0, 0)),
            pl.BlockSpec((1, 1), lambda i: (0, 0)),
        ],
        out_specs=pl.BlockSpec((RB, 1), lambda i: (i, 0)),
        out_shape=jax.ShapeDtypeStruct((B, 1), jnp.float32),
    )(e2d, ht, hstats, stats, lin2d, g1_r, b1_r, g2_c, b2_c, lb)


def kernel(x, emb_table, lin_table, lin_bias, bn1_gamma, bn1_beta, Wc, bc,
           bn2_gamma, bn2_beta):
    offs = jnp.arange(F, dtype=jnp.int32) * FIELD_DIM
    xo = (x + offs[None, :]).reshape(B * F)
    emb_lin = _sc_transpose(emb_table.T).reshape(V, D)
    emb_rows, lin_vals = _sc_gather(xo, emb_lin, lin_table.reshape(V))
    e2d = emb_rows.reshape(B, FD)
    lin2d = lin_vals.reshape(B, F)

    stats = _stats_call(e2d)
    ht, hstats = _mm_call(e2d, stats, Wc,
                          bc.reshape(F, 1),
                          bn1_gamma.reshape(1, F), bn1_beta.reshape(1, F))
    out2d = _final_call(e2d, ht, hstats, stats, lin2d,
                        bn1_gamma.reshape(1, F), bn1_beta.reshape(1, F),
                        bn2_gamma.reshape(F, 1), bn2_beta.reshape(F, 1),
                        lin_bias.reshape(1, 1))
    return out2d.reshape(B)
